# baseline (device time: 185963 ns/iter reference)
import jax
import jax.numpy as jnp
import numpy as np
from jax import lax
from jax.experimental import pallas as pl
from jax.experimental.pallas import tpu as pltpu

N_DEV = 4
B, SQ, SKV_SHARD, HQ, DH = 2, 512, 512, 8, 64
DM = 768
NCLS = 4
BLK = 64
CLS_ROWS = 128

_BLOCK_ORDER = [0, 4, 1, 5, 2, 6, 3, 7]
_PERM = np.concatenate([np.arange(BLK) + BLK * blk for blk in _BLOCK_ORDER])
_INV_PERM = np.argsort(_PERM)


def _body(x_ref, wq_ref, kv_ref, wo_ref, out_ref,
          q_ref, kvcomm, acc_ref, den_ref, send_sems, recv_sems):
    my = lax.axis_index("i")
    left = lax.rem(my - 1 + N_DEV, N_DEV)
    right = lax.rem(my + 1, N_DEV)

    barrier = pltpu.get_barrier_semaphore()
    for nbr in (left, right):
        pl.semaphore_signal(barrier, inc=1, device_id=(nbr,),
                            device_id_type=pl.DeviceIdType.MESH)
    pl.semaphore_wait(barrier, 2)

    send0 = pltpu.make_async_remote_copy(
        src_ref=kv_ref, dst_ref=kvcomm.at[0],
        send_sem=send_sems.at[0], recv_sem=recv_sems.at[0],
        device_id=(right,), device_id_type=pl.DeviceIdType.MESH)
    send0.start()

    for b in range(B):
        qb = jnp.dot(x_ref[b], wq_ref[...],
                     preferred_element_type=jnp.float32).astype(jnp.bfloat16)
        for h in range(HQ):
            q_ref[b * HQ + h] = qb[:, h * DH:(h + 1) * DH]

    acc_ref[...] = jnp.zeros(acc_ref.shape, acc_ref.dtype)
    den_ref[...] = jnp.zeros(den_ref.shape, den_ref.dtype)

    def process(kvr):
        def bh_body(bh, carry):
            q = q_ref[bh]
            k = kvr[0, bh]
            v = kvr[1, bh]
            ctx_parts = []
            den_parts = []
            for p in range(NCLS):
                sl = slice(p * CLS_ROWS, (p + 1) * CLS_ROWS)
                s = lax.dot_general(q[sl], k[sl], (((1,), (1,)), ((), ())),
                                    preferred_element_type=jnp.float32)
                w = jnp.exp(s * 0.125)
                ctx_parts.append(lax.dot_general(
                    w.astype(jnp.bfloat16), v[sl], (((1,), (0,)), ((), ())),
                    preferred_element_type=jnp.float32))
                den_parts.append(jnp.broadcast_to(
                    jnp.sum(w, axis=1, keepdims=True), (CLS_ROWS, DH)))
            acc_ref[bh] += jnp.concatenate(ctx_parts, axis=0)
            den_ref[bh] += jnp.concatenate(den_parts, axis=0)
            return carry
        lax.fori_loop(0, B * HQ, bh_body, 0)

    process(kv_ref)

    for c in range(1, N_DEV):
        recv = pltpu.make_async_remote_copy(
            src_ref=kv_ref, dst_ref=kvcomm.at[c - 1],
            send_sem=send_sems.at[c - 1], recv_sem=recv_sems.at[c - 1],
            device_id=(left,), device_id_type=pl.DeviceIdType.MESH)
        recv.wait_recv()
        if c < N_DEV - 1:
            send = pltpu.make_async_remote_copy(
                src_ref=kvcomm.at[c - 1], dst_ref=kvcomm.at[c],
                send_sem=send_sems.at[c], recv_sem=recv_sems.at[c],
                device_id=(right,), device_id_type=pl.DeviceIdType.MESH)
            send.start()
        process(kvcomm.at[c - 1])

    ctx = (acc_ref[...] / den_ref[...]).astype(jnp.bfloat16)
    for b in range(B):
        ctx_b = jnp.concatenate([ctx[b * HQ + h] for h in range(HQ)], axis=1)
        out_ref[b] = jnp.dot(ctx_b, wo_ref[...],
                             preferred_element_type=jnp.float32)

    for h in range(N_DEV - 1):
        src = kv_ref if h == 0 else kvcomm.at[h - 1]
        drain = pltpu.make_async_remote_copy(
            src_ref=src, dst_ref=kvcomm.at[h],
            send_sem=send_sems.at[h], recv_sem=recv_sems.at[h],
            device_id=(right,), device_id_type=pl.DeviceIdType.MESH)
        drain.wait_send()


def kernel(x, Wq, K_ext, V_ext, Wo):
    x2 = x[:, _PERM, :].astype(jnp.bfloat16)
    wq = Wq.astype(jnp.bfloat16)
    wo = Wo.astype(jnp.bfloat16)

    def prep(t):
        t = t.transpose(0, 2, 1, 3)[:, :, _PERM, :].astype(jnp.bfloat16)
        return t.reshape(B * HQ, SKV_SHARD, DH)

    kv = jnp.stack([prep(K_ext), prep(V_ext)], axis=0)

    out = pl.pallas_call(
        _body,
        out_shape=jax.ShapeDtypeStruct((B, SQ, DM), jnp.float32),
        in_specs=[pl.BlockSpec(memory_space=pltpu.VMEM)] * 4,
        out_specs=pl.BlockSpec(memory_space=pltpu.VMEM),
        scratch_shapes=[
            pltpu.VMEM((B * HQ, SQ, DH), jnp.bfloat16),
            pltpu.VMEM((N_DEV - 1, 2, B * HQ, SKV_SHARD, DH),
                       jnp.bfloat16),
            pltpu.VMEM((B * HQ, SQ, DH), jnp.float32),
            pltpu.VMEM((B * HQ, SQ, DH), jnp.float32),
            pltpu.SemaphoreType.DMA((N_DEV - 1,)),
            pltpu.SemaphoreType.DMA((N_DEV - 1,)),
        ],
        compiler_params=pltpu.CompilerParams(collective_id=0),
    )(x2, wq, kv, wo)
    return out[:, _INV_PERM, :]


# device time: 172806 ns/iter; 1.0761x vs baseline; 1.0761x over previous
import os

import jax
import jax.numpy as jnp
import numpy as np
from jax import lax
from jax.experimental import pallas as pl
from jax.experimental.pallas import tpu as pltpu

N_DEV = 4
B, SQ, SKV_SHARD, HQ, DH = 2, 512, 512, 8, 64
DM = 768
NCLS = 4
BLK = 64
CLS_ROWS = 128

_SKIP_COMPUTE = os.environ.get("ABLATE_SKIP_COMPUTE") == "1"
_LOCAL_ONLY = os.environ.get("ABLATE_LOCAL_ONLY") == "1"

_BLOCK_ORDER = [0, 4, 1, 5, 2, 6, 3, 7]
_PERM = np.concatenate([np.arange(BLK) + BLK * blk for blk in _BLOCK_ORDER])
_INV_PERM = np.argsort(_PERM)


def _body(x_ref, wq_ref, kv_ref, wo_ref, out_ref,
          q_ref, kvcomm, acc_ref, den_ref, send_sems, recv_sems):
    my = lax.axis_index("i")
    left = lax.rem(my - 1 + N_DEV, N_DEV)
    right = lax.rem(my + 1, N_DEV)

    if not _LOCAL_ONLY:
        barrier = pltpu.get_barrier_semaphore()
        for nbr in (left, right):
            pl.semaphore_signal(barrier, inc=1, device_id=(nbr,),
                                device_id_type=pl.DeviceIdType.MESH)
        pl.semaphore_wait(barrier, 2)

        send0 = pltpu.make_async_remote_copy(
            src_ref=kv_ref, dst_ref=kvcomm.at[0],
            send_sem=send_sems.at[0], recv_sem=recv_sems.at[0],
            device_id=(right,), device_id_type=pl.DeviceIdType.MESH)
        send0.start()

    for b in range(B):
        qb = jnp.dot(x_ref[b], wq_ref[...],
                     preferred_element_type=jnp.float32).astype(jnp.bfloat16)
        for h in range(HQ):
            q_ref[b * HQ + h] = qb[:, h * DH:(h + 1) * DH]

    acc_ref[...] = jnp.zeros(acc_ref.shape, acc_ref.dtype)
    den_ref[...] = jnp.zeros(den_ref.shape, den_ref.dtype)

    def process(kvr):
        if _SKIP_COMPUTE:
            return

        def bh_body(bh, carry):
            q = q_ref[bh]
            k = kvr[0, bh]
            v = kvr[1, bh]
            ctx_parts = []
            den_parts = []
            for p in range(NCLS):
                sl = slice(p * CLS_ROWS, (p + 1) * CLS_ROWS)
                s = lax.dot_general(q[sl], k[sl], (((1,), (1,)), ((), ())),
                                    preferred_element_type=jnp.float32)
                w = jnp.exp(s * 0.125)
                ctx_parts.append(lax.dot_general(
                    w.astype(jnp.bfloat16), v[sl], (((1,), (0,)), ((), ())),
                    preferred_element_type=jnp.float32))
                den_parts.append(jnp.broadcast_to(
                    jnp.sum(w, axis=1, keepdims=True), (CLS_ROWS, DH)))
            acc_ref[bh] += jnp.concatenate(ctx_parts, axis=0)
            den_ref[bh] += jnp.concatenate(den_parts, axis=0)
            return carry
        lax.fori_loop(0, B * HQ, bh_body, 0)

    process(kv_ref)

    if _LOCAL_ONLY:
        for _ in range(N_DEV - 1):
            process(kv_ref)

    for c in range(1, N_DEV) if not _LOCAL_ONLY else []:
        recv = pltpu.make_async_remote_copy(
            src_ref=kv_ref, dst_ref=kvcomm.at[c - 1],
            send_sem=send_sems.at[c - 1], recv_sem=recv_sems.at[c - 1],
            device_id=(left,), device_id_type=pl.DeviceIdType.MESH)
        recv.wait_recv()
        if c < N_DEV - 1:
            send = pltpu.make_async_remote_copy(
                src_ref=kvcomm.at[c - 1], dst_ref=kvcomm.at[c],
                send_sem=send_sems.at[c], recv_sem=recv_sems.at[c],
                device_id=(right,), device_id_type=pl.DeviceIdType.MESH)
            send.start()
        process(kvcomm.at[c - 1])

    ctx = (acc_ref[...] / den_ref[...]).astype(jnp.bfloat16)
    for b in range(B):
        ctx_b = jnp.concatenate([ctx[b * HQ + h] for h in range(HQ)], axis=1)
        out_ref[b] = jnp.dot(ctx_b, wo_ref[...],
                             preferred_element_type=jnp.float32)

    for h in range(N_DEV - 1) if not _LOCAL_ONLY else []:
        src = kv_ref if h == 0 else kvcomm.at[h - 1]
        drain = pltpu.make_async_remote_copy(
            src_ref=src, dst_ref=kvcomm.at[h],
            send_sem=send_sems.at[h], recv_sem=recv_sems.at[h],
            device_id=(right,), device_id_type=pl.DeviceIdType.MESH)
        drain.wait_send()


def kernel(x, Wq, K_ext, V_ext, Wo):
    x2 = x[:, _PERM, :].astype(jnp.bfloat16)
    wq = Wq.astype(jnp.bfloat16)
    wo = Wo.astype(jnp.bfloat16)

    def prep(t):
        t = t.transpose(0, 2, 1, 3)[:, :, _PERM, :].astype(jnp.bfloat16)
        return t.reshape(B * HQ, SKV_SHARD, DH)

    kv = jnp.stack([prep(K_ext), prep(V_ext)], axis=0)

    out = pl.pallas_call(
        _body,
        out_shape=jax.ShapeDtypeStruct((B, SQ, DM), jnp.float32),
        in_specs=[pl.BlockSpec(memory_space=pltpu.VMEM)] * 4,
        out_specs=pl.BlockSpec(memory_space=pltpu.VMEM),
        scratch_shapes=[
            pltpu.VMEM((B * HQ, SQ, DH), jnp.bfloat16),
            pltpu.VMEM((N_DEV - 1, 2, B * HQ, SKV_SHARD, DH),
                       jnp.bfloat16),
            pltpu.VMEM((B * HQ, SQ, DH), jnp.float32),
            pltpu.VMEM((B * HQ, SQ, DH), jnp.float32),
            pltpu.SemaphoreType.DMA((N_DEV - 1,)),
            pltpu.SemaphoreType.DMA((N_DEV - 1,)),
        ],
        compiler_params=pltpu.CompilerParams(collective_id=0),
    )(x2, wq, kv, wo)
    return out[:, _INV_PERM, :]


# device time: 105648 ns/iter; 1.7602x vs baseline; 1.6357x over previous
import os

import jax
import jax.numpy as jnp
import numpy as np
from jax import lax
from jax.experimental import pallas as pl
from jax.experimental.pallas import tpu as pltpu

N_DEV = 4
B, SQ, SKV_SHARD, HQ, DH = 2, 512, 512, 8, 64
DM = 768
NCLS = 4
BLK = 64
CLS_ROWS = 128

_SKIP_COMPUTE = os.environ.get("ABLATE_SKIP_COMPUTE") == "1"
_LOCAL_ONLY = os.environ.get("ABLATE_LOCAL_ONLY") == "1"

_BLOCK_ORDER = [0, 4, 1, 5, 2, 6, 3, 7]
_PERM = np.concatenate([np.arange(BLK) + BLK * blk for blk in _BLOCK_ORDER])
_INV_PERM = np.argsort(_PERM)


def _body(x_ref, wq_ref, kv_ref, wo_ref, out_ref,
          q_ref, kvcomm, acc_ref, den_ref, send_sems, recv_sems):
    my = lax.axis_index("i")
    left = lax.rem(my - 1 + N_DEV, N_DEV)
    right = lax.rem(my + 1, N_DEV)

    if not _LOCAL_ONLY:
        barrier = pltpu.get_barrier_semaphore()
        for nbr in (left, right):
            pl.semaphore_signal(barrier, inc=1, device_id=(nbr,),
                                device_id_type=pl.DeviceIdType.MESH)
        pl.semaphore_wait(barrier, 2)

        send0 = pltpu.make_async_remote_copy(
            src_ref=kv_ref, dst_ref=kvcomm.at[0],
            send_sem=send_sems.at[0], recv_sem=recv_sems.at[0],
            device_id=(right,), device_id_type=pl.DeviceIdType.MESH)
        send0.start()

    for b in range(B):
        q_ref[b] = jnp.dot(x_ref[b], wq_ref[...],
                           preferred_element_type=jnp.float32
                           ).astype(jnp.bfloat16)

    acc_ref[...] = jnp.zeros(acc_ref.shape, acc_ref.dtype)
    den_ref[...] = jnp.zeros(den_ref.shape, den_ref.dtype)

    def process(kvr):
        if _SKIP_COMPUTE:
            return
        for b in range(B):
            q = q_ref[b]
            k = kvr[0, b]
            v = kvr[1, b]
            for h in range(HQ):
                hs = slice(h * DH, (h + 1) * DH)
                ctx_parts = []
                den_parts = []
                for p in range(NCLS):
                    sl = slice(p * CLS_ROWS, (p + 1) * CLS_ROWS)
                    s = lax.dot_general(
                        q[sl, hs], k[sl, hs], (((1,), (1,)), ((), ())),
                        preferred_element_type=jnp.float32)
                    w = jnp.exp(s * 0.125)
                    ctx_parts.append(lax.dot_general(
                        w.astype(jnp.bfloat16), v[sl, hs],
                        (((1,), (0,)), ((), ())),
                        preferred_element_type=jnp.float32))
                    den_parts.append(jnp.broadcast_to(
                        jnp.sum(w, axis=1, keepdims=True), (CLS_ROWS, DH)))
                acc_ref[b, :, hs] += jnp.concatenate(ctx_parts, axis=0)
                den_ref[b, :, hs] += jnp.concatenate(den_parts, axis=0)

    process(kv_ref)

    if _LOCAL_ONLY:
        for _ in range(N_DEV - 1):
            process(kv_ref)

    for c in range(1, N_DEV) if not _LOCAL_ONLY else []:
        recv = pltpu.make_async_remote_copy(
            src_ref=kv_ref, dst_ref=kvcomm.at[c - 1],
            send_sem=send_sems.at[c - 1], recv_sem=recv_sems.at[c - 1],
            device_id=(left,), device_id_type=pl.DeviceIdType.MESH)
        recv.wait_recv()
        if c < N_DEV - 1:
            send = pltpu.make_async_remote_copy(
                src_ref=kvcomm.at[c - 1], dst_ref=kvcomm.at[c],
                send_sem=send_sems.at[c], recv_sem=recv_sems.at[c],
                device_id=(right,), device_id_type=pl.DeviceIdType.MESH)
            send.start()
        process(kvcomm.at[c - 1])

    ctx = (acc_ref[...] / den_ref[...]).astype(jnp.bfloat16)
    for b in range(B):
        out_ref[b] = jnp.dot(ctx[b], wo_ref[...],
                             preferred_element_type=jnp.float32)

    for h in range(N_DEV - 1) if not _LOCAL_ONLY else []:
        src = kv_ref if h == 0 else kvcomm.at[h - 1]
        drain = pltpu.make_async_remote_copy(
            src_ref=src, dst_ref=kvcomm.at[h],
            send_sem=send_sems.at[h], recv_sem=recv_sems.at[h],
            device_id=(right,), device_id_type=pl.DeviceIdType.MESH)
        drain.wait_send()


def kernel(x, Wq, K_ext, V_ext, Wo):
    x2 = x[:, _PERM, :].astype(jnp.bfloat16)
    wq = Wq.astype(jnp.bfloat16)
    wo = Wo.astype(jnp.bfloat16)

    def prep(t):
        return t.reshape(B, SKV_SHARD, HQ * DH)[:, _PERM, :].astype(jnp.bfloat16)

    kv = jnp.stack([prep(K_ext), prep(V_ext)], axis=0)

    out = pl.pallas_call(
        _body,
        out_shape=jax.ShapeDtypeStruct((B, SQ, DM), jnp.float32),
        in_specs=[pl.BlockSpec(memory_space=pltpu.VMEM)] * 4,
        out_specs=pl.BlockSpec(memory_space=pltpu.VMEM),
        scratch_shapes=[
            pltpu.VMEM((B, SQ, HQ * DH), jnp.bfloat16),
            pltpu.VMEM((N_DEV - 1, 2, B, SKV_SHARD, HQ * DH),
                       jnp.bfloat16),
            pltpu.VMEM((B, SQ, HQ * DH), jnp.float32),
            pltpu.VMEM((B, SQ, HQ * DH), jnp.float32),
            pltpu.SemaphoreType.DMA((N_DEV - 1,)),
            pltpu.SemaphoreType.DMA((N_DEV - 1,)),
        ],
        compiler_params=pltpu.CompilerParams(collective_id=0),
    )(x2, wq, kv, wo)
    return out[:, _INV_PERM, :]


# device time: 70371 ns/iter; 2.6426x vs baseline; 1.5013x over previous
import os

import jax
import jax.numpy as jnp
from jax import lax
from jax.experimental import pallas as pl
from jax.experimental.pallas import tpu as pltpu

N_DEV = 4
B, SQ, SKV_SHARD, HQ, DH = 2, 512, 512, 8, 64
DM = 768
NCLS = 4
BLK = 64

_SKIP_COMPUTE = os.environ.get("ABLATE_SKIP_COMPUTE") == "1"
_LOCAL_ONLY = os.environ.get("ABLATE_LOCAL_ONLY") == "1"


def _body(x_ref, wq_ref, k_ref, v_ref, wo_ref, out_ref,
          q_ref, kcomm, vcomm, acc_ref, den_ref, send_sems, recv_sems):
    my = lax.axis_index("i")
    left = lax.rem(my - 1 + N_DEV, N_DEV)
    right = lax.rem(my + 1, N_DEV)

    def rdma(src, dst, s_sem, r_sem, dev):
        return pltpu.make_async_remote_copy(
            src_ref=src, dst_ref=dst,
            send_sem=send_sems.at[s_sem], recv_sem=recv_sems.at[r_sem],
            device_id=(dev,), device_id_type=pl.DeviceIdType.MESH)

    sends = [
        (k_ref, kcomm.at[0], 0, 0, right),
        (k_ref, kcomm.at[1], 2, 2, left),
        (v_ref, vcomm.at[0], 1, 1, right),
        (v_ref, vcomm.at[1], 3, 3, left),
        (kcomm.at[0], kcomm.at[2], 4, 4, right),
        (vcomm.at[1], vcomm.at[2], 5, 5, left),
    ]

    if not _LOCAL_ONLY:
        barrier = pltpu.get_barrier_semaphore()
        for nbr in (left, right):
            pl.semaphore_signal(barrier, inc=1, device_id=(nbr,),
                                device_id_type=pl.DeviceIdType.MESH)
        pl.semaphore_wait(barrier, 2)

        for i in (0, 1, 2, 3):
            rdma(*sends[i]).start()

    for b in range(B):
        q_ref[b] = jnp.dot(x_ref[b], wq_ref[...],
                           preferred_element_type=jnp.float32
                           ).astype(jnp.bfloat16)

    acc_ref[...] = jnp.zeros(acc_ref.shape, acc_ref.dtype)
    den_ref[...] = jnp.zeros(den_ref.shape, den_ref.dtype)

    def process(kr, vr):
        if _SKIP_COMPUTE:
            return
        for b in range(B):
            q = q_ref[b]
            k = kr[b]
            v = vr[b]

            def cls(t, p, hs):
                lo = p * BLK
                hi = (p + 4) * BLK
                return jnp.concatenate(
                    [t[lo:lo + BLK, hs], t[hi:hi + BLK, hs]], axis=0)

            for h in range(HQ):
                hs = slice(h * DH, (h + 1) * DH)
                ctx_parts = []
                den_parts = []
                for p in range(NCLS):
                    s = lax.dot_general(
                        cls(q, p, hs), cls(k, p, hs),
                        (((1,), (1,)), ((), ())),
                        preferred_element_type=jnp.float32)
                    w = jnp.exp(s * 0.125)
                    ctx_parts.append(lax.dot_general(
                        w.astype(jnp.bfloat16), cls(v, p, hs),
                        (((1,), (0,)), ((), ())),
                        preferred_element_type=jnp.float32))
                    den_parts.append(jnp.broadcast_to(
                        jnp.sum(w, axis=1, keepdims=True), (2 * BLK, DH)))
                ctx = jnp.concatenate(
                    [ctx_parts[p][:BLK] for p in range(NCLS)]
                    + [ctx_parts[p][BLK:] for p in range(NCLS)], axis=0)
                den = jnp.concatenate(
                    [den_parts[p][:BLK] for p in range(NCLS)]
                    + [den_parts[p][BLK:] for p in range(NCLS)], axis=0)
                acc_ref[b, :, hs] += ctx
                den_ref[b, :, hs] += den

    process(k_ref, v_ref)

    if _LOCAL_ONLY:
        for _ in range(N_DEV - 1):
            process(k_ref, v_ref)
    else:
        rdma(*sends[0]).wait_recv()
        rdma(*sends[4]).start()
        rdma(*sends[2]).wait_recv()
        process(kcomm.at[0], vcomm.at[0])

        rdma(*sends[3]).wait_recv()
        rdma(*sends[5]).start()
        rdma(*sends[1]).wait_recv()
        process(kcomm.at[1], vcomm.at[1])

        rdma(*sends[4]).wait_recv()
        rdma(*sends[5]).wait_recv()
        process(kcomm.at[2], vcomm.at[2])

    ctx = (acc_ref[...] / den_ref[...]).astype(jnp.bfloat16)
    for b in range(B):
        out_ref[b] = jnp.dot(ctx[b], wo_ref[...],
                             preferred_element_type=jnp.float32)

    if not _LOCAL_ONLY:
        for i in range(6):
            rdma(*sends[i]).wait_send()


def kernel(x, Wq, K_ext, V_ext, Wo):
    x2 = x.astype(jnp.bfloat16)
    wq = Wq.astype(jnp.bfloat16)
    wo = Wo.astype(jnp.bfloat16)
    k2 = K_ext.reshape(B, SKV_SHARD, HQ * DH).astype(jnp.bfloat16)
    v2 = V_ext.reshape(B, SKV_SHARD, HQ * DH).astype(jnp.bfloat16)

    return pl.pallas_call(
        _body,
        out_shape=jax.ShapeDtypeStruct((B, SQ, DM), jnp.float32),
        in_specs=[pl.BlockSpec(memory_space=pltpu.VMEM)] * 5,
        out_specs=pl.BlockSpec(memory_space=pltpu.VMEM),
        scratch_shapes=[
            pltpu.VMEM((B, SQ, HQ * DH), jnp.bfloat16),
            pltpu.VMEM((3, B, SKV_SHARD, HQ * DH), jnp.bfloat16),
            pltpu.VMEM((3, B, SKV_SHARD, HQ * DH), jnp.bfloat16),
            pltpu.VMEM((B, SQ, HQ * DH), jnp.float32),
            pltpu.VMEM((B, SQ, HQ * DH), jnp.float32),
            pltpu.SemaphoreType.DMA((6,)),
            pltpu.SemaphoreType.DMA((6,)),
        ],
        compiler_params=pltpu.CompilerParams(collective_id=0),
    )(x2, wq, k2, v2, wo)


# device time: 32703 ns/iter; 5.6864x vs baseline; 2.1518x over previous
import jax
import jax.numpy as jnp
from jax import lax
from jax.experimental import pallas as pl
from jax.experimental.pallas import tpu as pltpu

N_DEV = 4
B, SQ, SKV_SHARD, HQ, DH = 2, 512, 512, 8, 64
DM = 768
NCLS = 4
BLK = 64
CR = 2 * BLK


def _body(x_ref, wq_ref, k_ref, v_ref, wo_ref, out_ref,
          kvout, kin, vin, ctxcomm, send_sems, recv_sems):
    my = lax.axis_index("i")
    left = lax.rem(my - 1 + N_DEV, N_DEV)
    right = lax.rem(my + 1, N_DEV)
    diag = lax.rem(my + 2, N_DEV)

    def rdma(src, dst, s_sem, r_sem, dev):
        return pltpu.make_async_remote_copy(
            src_ref=src, dst_ref=dst,
            send_sem=send_sems.at[s_sem], recv_sem=recv_sems.at[r_sem],
            device_id=(dev,), device_id_type=pl.DeviceIdType.MESH)

    barrier = pltpu.get_barrier_semaphore()
    for nbr in (left, right, diag):
        pl.semaphore_signal(barrier, inc=1, device_id=(nbr,),
                            device_id_type=pl.DeviceIdType.MESH)
    pl.semaphore_wait(barrier, 3)

    bf = jnp.bfloat16

    for j, dest in enumerate((left, right, diag)):
        for b in range(B):
            kvout[0, j, b, 0:BLK] = k_ref[b, pl.ds(dest * BLK, BLK)].astype(bf)
            kvout[0, j, b, BLK:CR] = (
                k_ref[b, pl.ds((dest + 4) * BLK, BLK)].astype(bf))
            kvout[1, j, b, 0:BLK] = v_ref[b, pl.ds(dest * BLK, BLK)].astype(bf)
            kvout[1, j, b, BLK:CR] = (
                v_ref[b, pl.ds((dest + 4) * BLK, BLK)].astype(bf))

    kv_sends = [
        (kvout.at[0, 0], kin.at[1], 0, 1, left),
        (kvout.at[0, 1], kin.at[0], 1, 0, right),
        (kvout.at[0, 2], kin.at[2], 2, 2, diag),
        (kvout.at[1, 0], vin.at[1], 3, 4, left),
        (kvout.at[1, 1], vin.at[0], 4, 3, right),
        (kvout.at[1, 2], vin.at[2], 5, 5, diag),
    ]
    for s in kv_sends:
        rdma(*s).start()

    wq = wq_ref[...].astype(bf)
    qcls = []
    for b in range(B):
        xb = jnp.concatenate(
            [x_ref[b, pl.ds(my * BLK, BLK)],
             x_ref[b, pl.ds((my + 4) * BLK, BLK)]], axis=0)
        qcls.append(jnp.dot(xb.astype(bf), wq,
                            preferred_element_type=jnp.float32
                            ).astype(bf))

    qh = [jnp.stack([qcls[b][:, h * DH:(h + 1) * DH] for h in range(HQ)])
          for b in range(B)]

    def attend(b, kb, vb):
        kh = jnp.stack([kb[:, h * DH:(h + 1) * DH] for h in range(HQ)])
        vh = jnp.stack([vb[:, h * DH:(h + 1) * DH] for h in range(HQ)])
        s = lax.dot_general(qh[b], kh, (((2,), (2,)), ((0,), (0,))),
                            preferred_element_type=jnp.float32)
        w = jnp.exp(s * 0.125)
        c = lax.dot_general(w.astype(bf), vh, (((2,), (1,)), ((0,), (0,))),
                            preferred_element_type=jnp.float32)
        return c, jnp.sum(w, axis=2, keepdims=True)

    acc = [None] * B
    den = [None] * B
    for b in range(B):
        k_loc = jnp.concatenate(
            [k_ref[b, pl.ds(my * BLK, BLK)].astype(bf),
             k_ref[b, pl.ds((my + 4) * BLK, BLK)].astype(bf)], axis=0)
        v_loc = jnp.concatenate(
            [v_ref[b, pl.ds(my * BLK, BLK)].astype(bf),
             v_ref[b, pl.ds((my + 4) * BLK, BLK)].astype(bf)], axis=0)
        acc[b], den[b] = attend(b, k_loc, v_loc)

    for slot, (ki, vi) in enumerate(((1, 4), (0, 3), (2, 5))):
        rdma(*kv_sends[ki]).wait_recv()
        rdma(*kv_sends[vi]).wait_recv()
        for b in range(B):
            c, d = attend(b, kin[slot, b], vin[slot, b])
            acc[b] += c
            den[b] += d

    for b in range(B):
        ctx = acc[b] / den[b]
        ctxcomm[my, b] = jnp.concatenate(
            [ctx[h] for h in range(HQ)], axis=1).astype(bf)

    ctx_sends = [
        (ctxcomm.at[my], ctxcomm.at[my], 6, 7, left),
        (ctxcomm.at[my], ctxcomm.at[my], 7, 6, right),
        (ctxcomm.at[my], ctxcomm.at[my], 8, 8, diag),
    ]
    for s in ctx_sends:
        rdma(*s).start()

    wo = wo_ref[...].astype(bf)

    def project(c):
        for b in range(B):
            o = jnp.dot(ctxcomm[c, b], wo,
                        preferred_element_type=jnp.float32)
            out_ref[b, pl.ds(c * BLK, BLK)] = o[0:BLK]
            out_ref[b, pl.ds((c + 4) * BLK, BLK)] = o[BLK:CR]

    project(my)
    for i, c in ((1, left), (0, right), (2, diag)):
        rdma(*ctx_sends[i]).wait_recv()
        project(c)

    for s in kv_sends + ctx_sends:
        rdma(*s).wait_send()


def kernel(x, Wq, K_ext, V_ext, Wo):
    k2 = K_ext.reshape(B, SKV_SHARD, HQ * DH)
    v2 = V_ext.reshape(B, SKV_SHARD, HQ * DH)

    return pl.pallas_call(
        _body,
        out_shape=jax.ShapeDtypeStruct((B, SQ, DM), jnp.float32),
        in_specs=[pl.BlockSpec(memory_space=pltpu.VMEM)] * 5,
        out_specs=pl.BlockSpec(memory_space=pltpu.VMEM),
        scratch_shapes=[
            pltpu.VMEM((2, 3, B, CR, HQ * DH), jnp.bfloat16),
            pltpu.VMEM((3, B, CR, HQ * DH), jnp.bfloat16),
            pltpu.VMEM((3, B, CR, HQ * DH), jnp.bfloat16),
            pltpu.VMEM((NCLS, B, CR, HQ * DH), jnp.bfloat16),
            pltpu.SemaphoreType.DMA((9,)),
            pltpu.SemaphoreType.DMA((9,)),
        ],
        compiler_params=pltpu.CompilerParams(collective_id=0),
    )(x, Wq, k2, v2, Wo)
